# baseline (speedup 1.0000x reference)
import jax
import jax.numpy as jnp
from jax import lax
from jax.experimental import pallas as pl
from jax.experimental.pallas import tpu as pltpu
from jax.experimental.pallas import tpu_sc as plsc

N = 10000
E = 320000
D = 128
H = D // 2
HW = H // 2
NC = 2
NS = 16
EPW = E // NS
NSUP = 10
S = EPW // NSUP
C = 64
HC = C // 2
SP = 2112
NCHN = SP // C
NROT = NCHN // 3
RPT = N // NS
ZFULL = RPT // C
ZREM = RPT % C
MASKHI = -65536


def _sc_body(rows4_hbm, cols4_hbm, vr4_hbm, vi4_hbm, xci_hbm,
             out_hbm,
             acc, rows2, gidx2, vrv, viv,
             xg0, xg1, xg2, og0, og1, og2, sg0, sg1, sg2, ss0, ss1, ss2):
    cid = lax.axis_index("c")
    sid = lax.axis_index("s")

    def _zero_buf(r, _):
        z = jnp.zeros((16,), jnp.float32)
        for j in range(8):
            og0[r, pl.ds(j * 16, 16)] = z
        return 0

    lax.fori_loop(0, C, _zero_buf, 0)
    zbase = sid * RPT
    for t in range(ZFULL):
        pltpu.sync_copy(og0, acc.at[pl.ds(zbase + t * C, C)])
    pltpu.sync_copy(og0.at[pl.ds(0, ZREM)],
                    acc.at[pl.ds(zbase + ZFULL * C, ZREM)])
    plsc.subcore_barrier()

    def _start_gather(jj, buf, sem):
        pltpu.async_copy(xci_hbm.at[gidx2.at[jj]], buf, sem)

    def _wait_gather(jj, buf, sem):
        pltpu.make_async_copy(xci_hbm.at[gidx2.at[jj]], buf, sem).wait()

    def _start_scatter(jj, buf, sem):
        pltpu.async_copy(buf, acc.at[rows2.at[jj]], sem, add=True)

    def _wait_scatter(jj, buf, sem):
        pltpu.make_async_copy(buf, acc.at[rows2.at[jj]], sem).wait()

    def _compute_half(jj, xg, og, h):
        def _edge(k, _):
            kk = jnp.full((16,), jj * C + k, jnp.int32)
            av = plsc.load_gather(vrv, [kk])
            bv = plsc.load_gather(viv, [kk])
            for g in range(2):
                wr = xg[k, pl.ds(g * 16, 16)]
                wi = xg[k, pl.ds(HW + g * 16, 16)]
                xr_e = plsc.bitcast(wr << 16, jnp.float32)
                xr_o = plsc.bitcast(wr & MASKHI, jnp.float32)
                xi_e = plsc.bitcast(wi << 16, jnp.float32)
                xi_o = plsc.bitcast(wi & MASKHI, jnp.float32)
                og[k, pl.ds(g * 32, 16)] = av * xr_e - bv * xi_e
                og[k, pl.ds(g * 32 + 16, 16)] = av * xr_o - bv * xi_o
                og[k, pl.ds(H + g * 32, 16)] = av * xi_e + bv * xr_e
                og[k, pl.ds(H + g * 32 + 16, 16)] = av * xi_o + bv * xr_o
            return 0

        lax.fori_loop(h * HC, (h + 1) * HC, _edge, 0, unroll=4)

    def _third(jj, xg_a, xg_c, og_a, og_c, sg_a, sg_c, ss_a, ss_c):
        _wait_gather(jj, xg_a, sg_a)
        _compute_half(jj, xg_a, og_a, 0)

        @pl.when(jj >= 1)
        def _():
            _wait_scatter(jj - 1, og_c, ss_c)

        @pl.when(jj + 2 < NCHN)
        def _():
            _start_gather(jj + 2, xg_c, sg_c)

        _compute_half(jj, xg_a, og_a, 1)
        _start_scatter(jj, og_a, ss_a)

    def _super(s, _):
        pltpu.sync_copy(rows4_hbm.at[sid, s], rows2)
        pltpu.sync_copy(cols4_hbm.at[sid, s], gidx2)
        pltpu.sync_copy(vr4_hbm.at[sid, s], vrv)
        pltpu.sync_copy(vi4_hbm.at[sid, s], viv)

        def _idx(jc, _):
            for i in range(0, C, 16):
                sl = pl.ds(i, 16)
                gidx2[jc, sl] = gidx2[jc, sl] * 2 + cid
            return 0

        lax.fori_loop(0, NCHN, _idx, 0)

        _start_gather(0, xg0, sg0)
        _start_gather(1, xg1, sg1)

        def _rot(r, _):
            j3 = 3 * r
            _third(j3, xg0, xg2, og0, og2, sg0, sg2, ss0, ss2)
            _third(j3 + 1, xg1, xg0, og1, og0, sg1, sg0, ss1, ss0)
            _third(j3 + 2, xg2, xg1, og2, og1, sg2, sg1, ss2, ss1)
            return 0

        lax.fori_loop(0, NROT, _rot, 0)
        _wait_scatter(NCHN - 1, og2, ss2)
        return 0

    lax.fori_loop(0, NSUP, _super, 0)
    plsc.subcore_barrier()

    for q in range(2):
        pltpu.sync_copy(
            acc.at[pl.ds(sid * RPT, RPT), pl.ds(q * H, H)],
            out_hbm.at[q, pl.ds(sid * RPT, RPT), pl.ds(cid * H, H)],
        )


@jax.jit
def kernel(L_values_real, L_values_imag, x_real, x_imag, L_indices):
    def _pad3(a):
        return jnp.pad(a.reshape(NS, NSUP, S), ((0, 0), (0, 0), (0, SP - S)))

    rows4 = _pad3(L_indices[0]).reshape(NS, NSUP, NCHN, C)
    cols4 = _pad3(L_indices[1]).reshape(NS, NSUP, NCHN, C)
    vr4 = _pad3(L_values_real)
    vi4 = _pad3(L_values_imag)
    xc = jnp.concatenate(
        [x_real.reshape(2 * N, H), x_imag.reshape(2 * N, H)], axis=1)
    xcp = xc.reshape(2 * N, 4, 2, 16).transpose(0, 1, 3, 2)
    xci = jax.lax.bitcast_convert_type(
        xcp.astype(jnp.bfloat16).reshape(2 * N, 2 * HW, 2), jnp.int32)

    mesh = plsc.VectorSubcoreMesh(
        core_axis_name="c", subcore_axis_name="s", num_cores=NC,
        num_subcores=NS)
    f = pl.kernel(
        _sc_body,
        out_type=jax.ShapeDtypeStruct((2, N, D), jnp.float32),
        mesh=mesh,
        compiler_params=pltpu.CompilerParams(use_tc_tiling_on_sc=False,
                                             needs_layout_passes=False),
        scratch_types=[
            pltpu.VMEM_SHARED((N, D), jnp.float32),
            pltpu.VMEM((NCHN, C), jnp.int32),
            pltpu.VMEM((NCHN, C), jnp.int32),
            pltpu.VMEM((SP,), jnp.float32),
            pltpu.VMEM((SP,), jnp.float32),
            pltpu.VMEM((C, 2 * HW), jnp.int32),
            pltpu.VMEM((C, 2 * HW), jnp.int32),
            pltpu.VMEM((C, 2 * HW), jnp.int32),
            pltpu.VMEM((C, D), jnp.float32),
            pltpu.VMEM((C, D), jnp.float32),
            pltpu.VMEM((C, D), jnp.float32),
            pltpu.SemaphoreType.DMA,
            pltpu.SemaphoreType.DMA,
            pltpu.SemaphoreType.DMA,
            pltpu.SemaphoreType.DMA,
            pltpu.SemaphoreType.DMA,
            pltpu.SemaphoreType.DMA,
        ],
    )
    return f(rows4, cols4, vr4, vi4, xci)


# speedup vs baseline: 2.8722x; 2.8722x over previous
import jax
import jax.numpy as jnp
from jax import lax
from jax.experimental import pallas as pl
from jax.experimental.pallas import tpu as pltpu
from jax.experimental.pallas import tpu_sc as plsc

N = 10000
E = 320000
D = 128
H = D // 2
NC = 2
NS = 16
EPW = E // NS
NSUP = 10
S = EPW // NSUP
C = 96
HC = C // 2
SP = 2016
NCHN = SP // C
NROT = NCHN // 3
RPT = N // NS
ZFULL = RPT // C
ZREM = RPT % C


def _sc_body(rows4_hbm, cols4_hbm, vr4_hbm, vi4_hbm, xc_hbm,
             out_hbm,
             acc, rows2, gidx2, vrv, viv,
             xg0, xg1, xg2, sg0, sg1, sg2, ss0, ss1, ss2):
    cid = lax.axis_index("c")
    sid = lax.axis_index("s")

    def _zero_buf(r, _):
        z = jnp.zeros((16,), jnp.float32)
        for j in range(8):
            xg0[r, pl.ds(j * 16, 16)] = z
        return 0

    lax.fori_loop(0, C, _zero_buf, 0)
    zbase = sid * RPT
    for t in range(ZFULL):
        pltpu.sync_copy(xg0, acc.at[pl.ds(zbase + t * C, C)])
    pltpu.sync_copy(xg0.at[pl.ds(0, ZREM)],
                    acc.at[pl.ds(zbase + ZFULL * C, ZREM)])
    plsc.subcore_barrier()

    def _start_gather(jj, buf, sem):
        pltpu.async_copy(xc_hbm.at[gidx2.at[jj]], buf, sem)

    def _wait_gather(jj, buf, sem):
        pltpu.make_async_copy(xc_hbm.at[gidx2.at[jj]], buf, sem).wait()

    def _start_scatter(jj, buf, sem):
        pltpu.async_copy(buf, acc.at[rows2.at[jj]], sem, add=True)

    def _wait_scatter(jj, buf, sem):
        pltpu.make_async_copy(buf, acc.at[rows2.at[jj]], sem).wait()

    def _compute_half(jj, buf, h):
        def _edge(k, _):
            kk = jnp.full((16,), jj * C + k, jnp.int32)
            av = plsc.load_gather(vrv, [kk])
            bv = plsc.load_gather(viv, [kk])
            for j in range(4):
                slr = pl.ds(j * 16, 16)
                sli = pl.ds(H + j * 16, 16)
                xr = buf[k, slr]
                xi = buf[k, sli]
                buf[k, slr] = av * xr - bv * xi
                buf[k, sli] = av * xi + bv * xr
            return 0

        lax.fori_loop(h * HC, (h + 1) * HC, _edge, 0, unroll=4)

    def _third(jj, buf_a, buf_c, sg_a, sg_c, ss_a, ss_c):
        _wait_gather(jj, buf_a, sg_a)
        _compute_half(jj, buf_a, 0)

        @pl.when(jj >= 1)
        def _():
            _wait_scatter(jj - 1, buf_c, ss_c)

        @pl.when(jj + 2 < NCHN)
        def _():
            _start_gather(jj + 2, buf_c, sg_c)

        _compute_half(jj, buf_a, 1)
        _start_scatter(jj, buf_a, ss_a)

    def _super(s, _):
        pltpu.sync_copy(rows4_hbm.at[sid, s], rows2)
        pltpu.sync_copy(cols4_hbm.at[sid, s], gidx2)
        pltpu.sync_copy(vr4_hbm.at[sid, s], vrv)
        pltpu.sync_copy(vi4_hbm.at[sid, s], viv)

        def _idx(jc, _):
            for i in range(C // 16):
                sl = pl.ds(i * 16, 16)
                gidx2[jc, sl] = gidx2[jc, sl] * 2 + cid
            return 0

        lax.fori_loop(0, NCHN, _idx, 0)

        _start_gather(0, xg0, sg0)
        _start_gather(1, xg1, sg1)

        def _rot(r, _):
            _third(3 * r, xg0, xg2, sg0, sg2, ss0, ss2)
            _third(3 * r + 1, xg1, xg0, sg1, sg0, ss1, ss0)
            _third(3 * r + 2, xg2, xg1, sg2, sg1, ss2, ss1)
            return 0

        lax.fori_loop(0, NROT, _rot, 0)
        _wait_scatter(NCHN - 1, xg2, ss2)
        return 0

    lax.fori_loop(0, NSUP, _super, 0)
    plsc.subcore_barrier()

    for q in range(2):
        pltpu.sync_copy(
            acc.at[pl.ds(sid * RPT, RPT), pl.ds(q * H, H)],
            out_hbm.at[q, pl.ds(sid * RPT, RPT), pl.ds(cid * H, H)],
        )


@jax.jit
def kernel(L_values_real, L_values_imag, x_real, x_imag, L_indices):
    def _pad3(a):
        return jnp.pad(a.reshape(NS, NSUP, S), ((0, 0), (0, 0), (0, SP - S)))

    rows4 = _pad3(L_indices[0]).reshape(NS, NSUP, NCHN, C)
    cols4 = _pad3(L_indices[1]).reshape(NS, NSUP, NCHN, C)
    vr4 = _pad3(L_values_real)
    vi4 = _pad3(L_values_imag)
    xc = jnp.concatenate(
        [x_real.reshape(2 * N, H), x_imag.reshape(2 * N, H)], axis=1)

    mesh = plsc.VectorSubcoreMesh(
        core_axis_name="c", subcore_axis_name="s", num_cores=NC,
        num_subcores=NS)
    f = pl.kernel(
        _sc_body,
        out_type=jax.ShapeDtypeStruct((2, N, D), jnp.float32),
        mesh=mesh,
        compiler_params=pltpu.CompilerParams(use_tc_tiling_on_sc=False,
                                             needs_layout_passes=False),
        scratch_types=[
            pltpu.VMEM_SHARED((N, D), jnp.float32),
            pltpu.VMEM((NCHN, C), jnp.int32),
            pltpu.VMEM((NCHN, C), jnp.int32),
            pltpu.VMEM((SP,), jnp.float32),
            pltpu.VMEM((SP,), jnp.float32),
            pltpu.VMEM((C, D), jnp.float32),
            pltpu.VMEM((C, D), jnp.float32),
            pltpu.VMEM((C, D), jnp.float32),
            pltpu.SemaphoreType.DMA,
            pltpu.SemaphoreType.DMA,
            pltpu.SemaphoreType.DMA,
            pltpu.SemaphoreType.DMA,
            pltpu.SemaphoreType.DMA,
            pltpu.SemaphoreType.DMA,
        ],
    )
    return f(rows4, cols4, vr4, vi4, xc)
